# baseline (speedup 1.0000x reference)
import functools

import jax
import jax.numpy as jnp
from jax import lax
from jax.experimental import pallas as pl
from jax.experimental.pallas import tpu as pltpu
from jax.experimental.pallas import tpu_sc as plsc

N = 10000
E = 320000
D = 128

NS = 16
NC = 2
CORE_N = 5120
R_ACC = 5248
TRASH = 5200
OROWS2 = CORE_N // NS

CHUNK = 128
EPT = 20480
NCHUNK = EPT // CHUNK
E_PAD = NS * EPT
N_PAD = 10240
ZROWS = N_PAD // NS
OROWS = N_PAD // NS


def _sc_segment_sum(x, src_idx, dst_idx):
    mesh = plsc.VectorSubcoreMesh(
        core_axis_name="c", subcore_axis_name="s", num_cores=NC,
        num_subcores=NS)

    @functools.partial(
        pl.kernel,
        out_type=(
            jax.ShapeDtypeStruct((N_PAD, D), jnp.float32),
            jax.ShapeDtypeStruct((N_PAD,), jnp.float32),
        ),
        mesh=mesh,
        scratch_types=(
            pltpu.VMEM((CHUNK,), jnp.int32),
            pltpu.VMEM((CHUNK,), jnp.int32),
            pltpu.VMEM((CHUNK,), jnp.int32),
            pltpu.VMEM((CHUNK,), jnp.int32),
            pltpu.VMEM((CHUNK, D), jnp.float32),
            pltpu.VMEM((CHUNK, D), jnp.float32),
            pltpu.VMEM((CHUNK,), jnp.float32),
            pltpu.VMEM((3 * CHUNK,), jnp.float32),
            pltpu.VMEM((2 * CORE_N // NS,), jnp.float32),
            pltpu.VMEM_SHARED((R_ACC, D), jnp.float32),
            pltpu.VMEM_SHARED((R_ACC,), jnp.float32),
            pltpu.SemaphoreType.DMA,
            pltpu.SemaphoreType.DMA,
            pltpu.SemaphoreType.DMA,
            pltpu.SemaphoreType.DMA,
            pltpu.SemaphoreType.DMA,
            pltpu.SemaphoreType.DMA,
            pltpu.SemaphoreType.DMA,
            pltpu.SemaphoreType.DMA,
        ),
    )
    def sc_kernel(x_hbm, src_hbm, dst_hbm, psum_hbm, pcnt_hbm,
                  srcc0, srcc1, dstc0, dstc1, rows0_v, rows1_v, ones_v, z1_v,
                  cstage_v, acc_s, cacc_s, semi0, semi1, semg0, semg1,
                  sems0, sems1, sems2, sems3):
        cid = lax.axis_index("c")
        sid = lax.axis_index("s")

        zeros16 = jnp.zeros((16,), jnp.float32)
        ones16 = jnp.ones((16,), jnp.float32)

        for i in range(CHUNK):
            for j in range(D // 16):
                rows0_v[i, pl.ds(j * 16, 16)] = zeros16
        for j in range(CHUNK // 16):
            ones_v[pl.ds(j * 16, 16)] = ones16
        for j in range(3 * CHUNK // 16):
            z1_v[pl.ds(j * 16, 16)] = zeros16

        zbase = pl.multiple_of(
            jnp.minimum(sid * (R_ACC // NS), R_ACC - 3 * CHUNK), 8)
        for t in range(3):
            pltpu.async_copy(rows0_v, acc_s.at[pl.ds(zbase + t * CHUNK, CHUNK)],
                             sems0)
        pltpu.async_copy(z1_v, cacc_s.at[pl.ds(zbase, 3 * CHUNK)], sems1)
        for t in range(3):
            pltpu.make_async_copy(
                rows0_v, acc_s.at[pl.ds(zbase + t * CHUNK, CHUNK)], sems0).wait()
        pltpu.make_async_copy(z1_v, cacc_s.at[pl.ds(zbase, 3 * CHUNK)], sems1).wait()

        plsc.subcore_barrier()

        ebase = sid * EPT

        dbase = cid * E_PAD + ebase

        def fire_idx(c, sbuf, dbuf, sem):
            pltpu.async_copy(src_hbm.at[pl.ds(ebase + c * CHUNK, CHUNK)], sbuf, sem)
            pltpu.async_copy(dst_hbm.at[pl.ds(dbase + c * CHUNK, CHUNK)], dbuf, sem)

        def wait_idx(sbuf, dbuf, sem):
            pltpu.make_async_copy(src_hbm.at[pl.ds(0, CHUNK)], sbuf, sem).wait()
            pltpu.make_async_copy(dst_hbm.at[pl.ds(0, CHUNK)], dbuf, sem).wait()

        def fire_rows(sbuf, buf, sem):
            pltpu.async_copy(x_hbm.at[sbuf], buf, sem)

        def wait_rows(buf, sem):
            pltpu.make_async_copy(x_hbm.at[pl.ds(0, CHUNK)], buf, sem).wait()

        def scat(dbuf, buf):
            pltpu.sync_copy(buf, acc_s.at[dbuf], add=True)
            pltpu.sync_copy(ones_v, cacc_s.at[dbuf], add=True)

        fire_idx(0, srcc0, dstc0, semi0)
        fire_idx(1, srcc1, dstc1, semi1)
        wait_idx(srcc0, dstc0, semi0)
        fire_rows(srcc0, rows0_v, semg0)

        def pair_body(i, _):
            c0 = 2 * i
            wait_idx(srcc1, dstc1, semi1)
            fire_rows(srcc1, rows1_v, semg1)
            wait_rows(rows0_v, semg0)
            scat(dstc0, rows0_v)
            nxt2 = jnp.where(c0 + 2 < NCHUNK, c0 + 2, 0)
            fire_idx(nxt2, srcc0, dstc0, semi0)
            wait_idx(srcc0, dstc0, semi0)
            fire_rows(srcc0, rows0_v, semg0)
            wait_rows(rows1_v, semg1)
            scat(dstc1, rows1_v)
            nxt3 = jnp.where(c0 + 3 < NCHUNK, c0 + 3, 0)
            fire_idx(nxt3, srcc1, dstc1, semi1)
            return 0

        lax.fori_loop(0, NCHUNK // 2, pair_body, 0)
        wait_rows(rows0_v, semg0)
        wait_idx(srcc1, dstc1, semi1)

        plsc.subcore_barrier()

        lbase = sid * OROWS2
        gbase = cid * CORE_N + sid * OROWS2
        pltpu.sync_copy(acc_s.at[pl.ds(lbase, OROWS2)],
                        psum_hbm.at[pl.ds(gbase, OROWS2)])
        cbase = pl.multiple_of(
            jnp.minimum(sid * OROWS2, CORE_N - 2 * OROWS2), 64)
        pltpu.sync_copy(cacc_s.at[pl.ds(cbase, 2 * OROWS2)], cstage_v)
        pltpu.sync_copy(cstage_v,
                        pcnt_hbm.at[pl.ds(cid * CORE_N + cbase, 2 * OROWS2)])

    return sc_kernel(x, src_idx, dst_idx)


def _tc_tail_body(x_ref, psum_ref, pcnt_ref, wl_ref, wr_ref, wlt_ref, wrt_ref,
                  out_t_ref, out_x_ref):
    B = x_ref.shape[0]
    summed = psum_ref[...]
    cnt = pcnt_ref[...]
    neigh = summed / jnp.maximum(cnt, 1.0)
    xb = x_ref[...]
    self_x = jnp.dot(xb, wl_ref[...], preferred_element_type=jnp.float32)
    self_s = jnp.dot(xb, wlt_ref[...], preferred_element_type=jnp.float32)
    neigh_x = jnp.dot(neigh, wr_ref[...], preferred_element_type=jnp.float32)
    neigh_s = jnp.dot(neigh, wrt_ref[...], preferred_element_type=jnp.float32)
    out_t_ref[...] = self_s + neigh_s
    out_x_ref[...] = jax.nn.sigmoid(self_x + neigh_x)


def _tc_tail(x_pad, psum, pcnt_col, wl_t, wr_t, wlt_t, wrt_t):
    B = 1024
    grid = (N_PAD // B,)
    return pl.pallas_call(
        _tc_tail_body,
        grid=grid,
        in_specs=[
            pl.BlockSpec((B, D), lambda i: (i, 0)),
            pl.BlockSpec((B, D), lambda i: (i, 0)),
            pl.BlockSpec((B, 1), lambda i: (i, 0)),
            pl.BlockSpec((D, D), lambda i: (0, 0)),
            pl.BlockSpec((D, D), lambda i: (0, 0)),
            pl.BlockSpec((D, D), lambda i: (0, 0)),
            pl.BlockSpec((D, D), lambda i: (0, 0)),
        ],
        out_specs=[
            pl.BlockSpec((B, D), lambda i: (i, 0)),
            pl.BlockSpec((B, D), lambda i: (i, 0)),
        ],
        out_shape=[
            jax.ShapeDtypeStruct((N_PAD, D), jnp.float32),
            jax.ShapeDtypeStruct((N_PAD, D), jnp.float32),
        ],
    )(x_pad, psum, pcnt_col, wl_t, wr_t, wlt_t, wrt_t)


@jax.jit
def kernel(x, edge_index, W_l, W_r, W_lt, W_rt):
    src = edge_index[0]
    dst = edge_index[1]
    pad = E_PAD - E
    src_p = jnp.concatenate([src, jnp.zeros((pad,), jnp.int32)])
    dst_p = jnp.concatenate([dst, jnp.full((pad,), N, jnp.int32)])
    dst0 = jnp.where(dst_p < CORE_N, dst_p, TRASH)
    dst1 = jnp.where(dst_p >= CORE_N, dst_p - CORE_N, TRASH)
    dst01 = jnp.concatenate([dst0, dst1])

    psum, pcnt = _sc_segment_sum(x, src_p, dst01)
    x_pad = jnp.concatenate([x, jnp.zeros((N_PAD - N, D), jnp.float32)])
    out_t, out_x = _tc_tail(x_pad, psum, pcnt[:, None],
                            W_l.T, W_r.T, W_lt.T, W_rt.T)
    return (out_t[:N], out_x[:N])


# speedup vs baseline: 1.6011x; 1.6011x over previous
import functools

import jax
import jax.numpy as jnp
from jax import lax
from jax.experimental import pallas as pl
from jax.experimental.pallas import tpu as pltpu
from jax.experimental.pallas import tpu_sc as plsc

N = 10000
E = 320000
D = 128

NS = 16

CHUNK = 128
EPT = 20480
NCHUNK = EPT // CHUNK
E_PAD = NS * EPT
N_PAD = 10240
ZROWS = N_PAD // NS
OROWS = N_PAD // NS


def _sc_segment_sum(x, src_idx, dst_idx):
    mesh = plsc.VectorSubcoreMesh(
        core_axis_name="c", subcore_axis_name="s", num_cores=1,
        num_subcores=NS)

    @functools.partial(
        pl.kernel,
        out_type=(
            jax.ShapeDtypeStruct((N_PAD, D), jnp.float32),
            jax.ShapeDtypeStruct((N_PAD,), jnp.float32),
        ),
        mesh=mesh,
        scratch_types=(
            pltpu.VMEM((CHUNK,), jnp.int32),
            pltpu.VMEM((CHUNK,), jnp.int32),
            pltpu.VMEM((CHUNK,), jnp.int32),
            pltpu.VMEM((CHUNK,), jnp.int32),
            pltpu.VMEM((CHUNK, D), jnp.float32),
            pltpu.VMEM((CHUNK, D), jnp.float32),
            pltpu.VMEM((CHUNK,), jnp.float32),
            pltpu.VMEM((ZROWS,), jnp.float32),
            pltpu.VMEM_SHARED((N_PAD, D), jnp.float32),
            pltpu.VMEM_SHARED((N_PAD,), jnp.float32),
            pltpu.SemaphoreType.DMA,
            pltpu.SemaphoreType.DMA,
            pltpu.SemaphoreType.DMA,
            pltpu.SemaphoreType.DMA,
            pltpu.SemaphoreType.DMA,
            pltpu.SemaphoreType.DMA,
            pltpu.SemaphoreType.DMA,
            pltpu.SemaphoreType.DMA,
        ),
    )
    def sc_kernel(x_hbm, src_hbm, dst_hbm, psum_hbm, pcnt_hbm,
                  srcc0, srcc1, dstc0, dstc1, rows0_v, rows1_v, ones_v, z1_v,
                  acc_s, cacc_s, semi0, semi1, semg0, semg1,
                  sems0, sems1, sems2, sems3):
        sid = lax.axis_index("s")

        zeros16 = jnp.zeros((16,), jnp.float32)
        ones16 = jnp.ones((16,), jnp.float32)

        for i in range(CHUNK):
            for j in range(D // 16):
                rows0_v[i, pl.ds(j * 16, 16)] = zeros16
        for j in range(CHUNK // 16):
            ones_v[pl.ds(j * 16, 16)] = ones16
        for j in range(ZROWS // 16):
            z1_v[pl.ds(j * 16, 16)] = zeros16

        zbase = sid * ZROWS
        for t in range(ZROWS // CHUNK):
            pltpu.async_copy(rows0_v, acc_s.at[pl.ds(zbase + t * CHUNK, CHUNK)],
                             sems0)
        pltpu.async_copy(z1_v, cacc_s.at[pl.ds(zbase, ZROWS)], sems1)
        for t in range(ZROWS // CHUNK):
            pltpu.make_async_copy(
                rows0_v, acc_s.at[pl.ds(zbase + t * CHUNK, CHUNK)], sems0).wait()
        pltpu.make_async_copy(z1_v, cacc_s.at[pl.ds(zbase, ZROWS)], sems1).wait()

        plsc.subcore_barrier()

        ebase = sid * EPT

        def fire_idx(c, sbuf, dbuf, sem):
            off = ebase + c * CHUNK
            pltpu.async_copy(src_hbm.at[pl.ds(off, CHUNK)], sbuf, sem)
            pltpu.async_copy(dst_hbm.at[pl.ds(off, CHUNK)], dbuf, sem)

        def wait_idx(sbuf, dbuf, sem):
            pltpu.make_async_copy(src_hbm.at[pl.ds(0, CHUNK)], sbuf, sem).wait()
            pltpu.make_async_copy(dst_hbm.at[pl.ds(0, CHUNK)], dbuf, sem).wait()

        def fire_rows(sbuf, buf, sem):
            pltpu.async_copy(x_hbm.at[sbuf], buf, sem)

        def wait_rows(buf, sem):
            pltpu.make_async_copy(x_hbm.at[pl.ds(0, CHUNK)], buf, sem).wait()

        def scat(dbuf, buf):
            pltpu.sync_copy(buf, acc_s.at[dbuf], add=True)
            pltpu.sync_copy(ones_v, cacc_s.at[dbuf], add=True)

        fire_idx(0, srcc0, dstc0, semi0)
        fire_idx(1, srcc1, dstc1, semi1)
        wait_idx(srcc0, dstc0, semi0)
        fire_rows(srcc0, rows0_v, semg0)

        def pair_body(i, _):
            c0 = 2 * i
            wait_idx(srcc1, dstc1, semi1)
            fire_rows(srcc1, rows1_v, semg1)
            wait_rows(rows0_v, semg0)
            scat(dstc0, rows0_v)
            nxt2 = jnp.where(c0 + 2 < NCHUNK, c0 + 2, 0)
            fire_idx(nxt2, srcc0, dstc0, semi0)
            wait_idx(srcc0, dstc0, semi0)
            fire_rows(srcc0, rows0_v, semg0)
            wait_rows(rows1_v, semg1)
            scat(dstc1, rows1_v)
            nxt3 = jnp.where(c0 + 3 < NCHUNK, c0 + 3, 0)
            fire_idx(nxt3, srcc1, dstc1, semi1)
            return 0

        lax.fori_loop(0, NCHUNK // 2, pair_body, 0)
        wait_rows(rows0_v, semg0)
        wait_idx(srcc1, dstc1, semi1)

        plsc.subcore_barrier()

        obase = sid * OROWS
        pltpu.sync_copy(acc_s.at[pl.ds(obase, OROWS)],
                        psum_hbm.at[pl.ds(obase, OROWS)])
        pltpu.sync_copy(cacc_s.at[pl.ds(obase, OROWS)],
                        pcnt_hbm.at[pl.ds(obase, OROWS)])

    return sc_kernel(x, src_idx, dst_idx)


def _tc_tail_body(x_ref, psum_ref, pcnt_ref, wl_ref, wr_ref, wlt_ref, wrt_ref,
                  out_t_ref, out_x_ref):
    B = x_ref.shape[0]
    summed = psum_ref[...]
    cnt = pcnt_ref[...]
    neigh = summed / jnp.maximum(cnt, 1.0)
    xb = x_ref[...]
    self_x = jnp.dot(xb, wl_ref[...], preferred_element_type=jnp.float32)
    self_s = jnp.dot(xb, wlt_ref[...], preferred_element_type=jnp.float32)
    neigh_x = jnp.dot(neigh, wr_ref[...], preferred_element_type=jnp.float32)
    neigh_s = jnp.dot(neigh, wrt_ref[...], preferred_element_type=jnp.float32)
    out_t_ref[...] = self_s + neigh_s
    out_x_ref[...] = jax.nn.sigmoid(self_x + neigh_x)


def _tc_tail(x_pad, psum, pcnt_col, wl_t, wr_t, wlt_t, wrt_t):
    B = 1024
    grid = (N_PAD // B,)
    return pl.pallas_call(
        _tc_tail_body,
        grid=grid,
        in_specs=[
            pl.BlockSpec((B, D), lambda i: (i, 0)),
            pl.BlockSpec((B, D), lambda i: (i, 0)),
            pl.BlockSpec((B, 1), lambda i: (i, 0)),
            pl.BlockSpec((D, D), lambda i: (0, 0)),
            pl.BlockSpec((D, D), lambda i: (0, 0)),
            pl.BlockSpec((D, D), lambda i: (0, 0)),
            pl.BlockSpec((D, D), lambda i: (0, 0)),
        ],
        out_specs=[
            pl.BlockSpec((B, D), lambda i: (i, 0)),
            pl.BlockSpec((B, D), lambda i: (i, 0)),
        ],
        out_shape=[
            jax.ShapeDtypeStruct((N_PAD, D), jnp.float32),
            jax.ShapeDtypeStruct((N_PAD, D), jnp.float32),
        ],
    )(x_pad, psum, pcnt_col, wl_t, wr_t, wlt_t, wrt_t)


@jax.jit
def kernel(x, edge_index, W_l, W_r, W_lt, W_rt):
    src = edge_index[0]
    dst = edge_index[1]
    pad = E_PAD - E
    src_p = jnp.concatenate([src, jnp.zeros((pad,), jnp.int32)])
    dst_p = jnp.concatenate([dst, jnp.full((pad,), N, jnp.int32)])

    psum, pcnt = _sc_segment_sum(x, src_p, dst_p)
    x_pad = jnp.concatenate([x, jnp.zeros((N_PAD - N, D), jnp.float32)])
    out_t, out_x = _tc_tail(x_pad, psum, pcnt[:, None],
                            W_l.T, W_r.T, W_lt.T, W_rt.T)
    return (out_t[:N], out_x[:N])
